# Initial kernel scaffold; baseline (speedup 1.0000x reference)
import jax
import jax.numpy as jnp
from jax.experimental import pallas as pl


def kernel(x, use_quantized, W):
    raise NotImplementedError("write your pallas kernel here")


# speedup vs baseline: 174.0834x; 174.0834x over previous
import functools

import jax
import jax.numpy as jnp
from jax import lax
from jax.experimental import pallas as pl
from jax.experimental.pallas import tpu as pltpu
from jax.experimental.pallas import tpu_sc as plsc

B = 16384
L = 200
D = 10
NC = 2
NS = 16
NW = NC * NS
RPW = B // NW
NGROUPS = RPW // 16

_mesh = plsc.VectorSubcoreMesh(
    core_axis_name="c", subcore_axis_name="s", num_cores=NC, num_subcores=NS
)


@functools.partial(
    pl.kernel,
    out_type=jax.ShapeDtypeStruct((B, D), jnp.float32),
    mesh=_mesh,
    scratch_types=[
        pltpu.VMEM((RPW, L), jnp.int32),
        pltpu.VMEM((RPW, D), jnp.float32),
        pltpu.VMEM((D, 16), jnp.float32),
        pltpu.VMEM((D, 16), jnp.float32),
        pltpu.SemaphoreType.DMA,
    ],
    compiler_params=pltpu.CompilerParams(
        needs_layout_passes=False, use_tc_tiling_on_sc=False
    ),
)
def _bag_mean(x_hbm, w0_hbm, w1_hbm, out_hbm, x_v, out_v, w0_v, w1_v, sem):
    wid = lax.axis_index("s") * NC + lax.axis_index("c")
    base = wid * RPW
    cp = pltpu.async_copy(x_hbm.at[pl.ds(base, RPW), :], x_v, sem)
    pltpu.sync_copy(w0_hbm, w0_v)
    pltpu.sync_copy(w1_hbm, w1_v)

    w0b = []
    cfb = []
    inv_l = jnp.float32(1.0 / L)
    for d in range(D):
        w0d = w0_v[d, :]
        w1d = w1_v[d, :]
        w0b.append(w0d)
        cfb.append((w1d - w0d) * inv_l)

    cp.wait()

    iota16 = lax.iota(jnp.int32, 16)

    def group_body(g, carry):
        rows = jnp.full((16,), g * 16, jnp.int32) + iota16

        def col_body(l, acc):
            col = jnp.full((16,), l, jnp.int32)
            return acc + plsc.load_gather(x_v, [rows, col])

        acc = lax.fori_loop(
            0, L, col_body, jnp.zeros((16,), jnp.int32), unroll=8
        )
        s = acc.astype(jnp.float32)
        for d in range(D):
            vals = w0b[d] + s * cfb[d]
            plsc.store_scatter(out_v, [rows, jnp.full((16,), d, jnp.int32)], vals)
        return carry

    lax.fori_loop(0, NGROUPS, group_body, 0)
    pltpu.sync_copy(out_v, out_hbm.at[pl.ds(base, RPW), :])


def kernel(x, use_quantized, W):
    del use_quantized
    x = x.astype(jnp.int32)
    Wf = W.astype(jnp.float32)
    w0 = jnp.broadcast_to(Wf[0][:, None], (D, 16))
    w1 = jnp.broadcast_to(Wf[1][:, None], (D, 16))
    return _bag_mean(x, w0, w1)


# speedup vs baseline: 174.7066x; 1.0036x over previous
import functools

import jax
import jax.numpy as jnp
from jax import lax
from jax.experimental import pallas as pl
from jax.experimental.pallas import tpu as pltpu
from jax.experimental.pallas import tpu_sc as plsc

B = 16384
L = 200
D = 10
NC = 2
NS = 16
NW = NC * NS
RPW = B // NW
NGROUPS = RPW // 16

_mesh = plsc.VectorSubcoreMesh(
    core_axis_name="c", subcore_axis_name="s", num_cores=NC, num_subcores=NS
)


@functools.partial(
    pl.kernel,
    out_type=jax.ShapeDtypeStruct((B, D), jnp.float32),
    mesh=_mesh,
    scratch_types=[
        pltpu.VMEM((RPW * L,), jnp.int32),
        pltpu.VMEM((RPW, D), jnp.float32),
        pltpu.VMEM((D, 16), jnp.float32),
        pltpu.VMEM((D, 16), jnp.float32),
        pltpu.SemaphoreType.DMA,
    ],
    compiler_params=pltpu.CompilerParams(
        needs_layout_passes=False, use_tc_tiling_on_sc=False
    ),
)
def _bag_mean(x_hbm, w0_hbm, w1_hbm, out_hbm, x_v, out_v, w0_v, w1_v, sem):
    wid = lax.axis_index("s") * NC + lax.axis_index("c")
    base = wid * RPW
    cp = pltpu.async_copy(x_hbm.at[pl.ds(base * L, RPW * L)], x_v, sem)
    pltpu.sync_copy(w0_hbm, w0_v)
    pltpu.sync_copy(w1_hbm, w1_v)

    w0b = []
    cfb = []
    inv_l = jnp.float32(1.0 / L)
    for d in range(D):
        w0d = w0_v[d, :]
        w1d = w1_v[d, :]
        w0b.append(w0d)
        cfb.append((w1d - w0d) * inv_l)

    cp.wait()

    iota16 = lax.iota(jnp.int32, 16)

    def group_body(g, carry):
        rows = jnp.full((16,), g * 16, jnp.int32) + iota16

        def col_body(_, c):
            acc, idx = c
            return acc + plsc.load_gather(x_v, [idx]), idx + 1

        acc, _ = lax.fori_loop(
            0,
            L,
            col_body,
            (jnp.zeros((16,), jnp.int32), rows * jnp.int32(L)),
            unroll=25,
        )
        s = acc.astype(jnp.float32)
        for d in range(D):
            vals = w0b[d] + s * cfb[d]
            plsc.store_scatter(out_v, [rows, jnp.full((16,), d, jnp.int32)], vals)
        return carry

    lax.fori_loop(0, NGROUPS, group_body, 0)
    pltpu.sync_copy(out_v, out_hbm.at[pl.ds(base, RPW), :])


def kernel(x, use_quantized, W):
    del use_quantized
    x = x.astype(jnp.int32).reshape(-1)
    Wf = W.astype(jnp.float32)
    w0 = jnp.broadcast_to(Wf[0][:, None], (D, 16))
    w1 = jnp.broadcast_to(Wf[1][:, None], (D, 16))
    return _bag_mean(x, w0, w1)


# speedup vs baseline: 465.1139x; 2.6623x over previous
import functools

import jax
import jax.numpy as jnp
from jax import lax
from jax.experimental import pallas as pl
from jax.experimental.pallas import tpu as pltpu
from jax.experimental.pallas import tpu_sc as plsc

B = 16384
L = 200
D = 10
NC = 2
NS = 16
NW = NC * NS
BPW = B // NW
NGROUPS = BPW // 16

_mesh = plsc.VectorSubcoreMesh(
    core_axis_name="c", subcore_axis_name="s", num_cores=NC, num_subcores=NS
)


@functools.partial(
    pl.kernel,
    out_type=jax.ShapeDtypeStruct((D, B), jnp.float32),
    mesh=_mesh,
    scratch_types=[
        pltpu.VMEM((L, BPW), jnp.int32),
        pltpu.VMEM((D, BPW), jnp.float32),
        pltpu.VMEM((D * 16,), jnp.float32),
        pltpu.VMEM((D * 16,), jnp.float32),
        pltpu.SemaphoreType.DMA,
    ],
    compiler_params=pltpu.CompilerParams(
        needs_layout_passes=False, use_tc_tiling_on_sc=True
    ),
)
def _bag_mean(xt_hbm, w0_hbm, w1_hbm, out_hbm, x_v, out_v, w0_v, w1_v, sem):
    wid = lax.axis_index("s") * NC + lax.axis_index("c")
    b0 = wid * BPW
    cp = pltpu.async_copy(xt_hbm.at[:, pl.ds(b0, BPW)], x_v, sem)
    pltpu.sync_copy(w0_hbm, w0_v)
    pltpu.sync_copy(w1_hbm, w1_v)

    w0b = []
    cfb = []
    inv_l = jnp.float32(1.0 / L)
    for d in range(D):
        w0d = w0_v[pl.ds(16 * d, 16)]
        w1d = w1_v[pl.ds(16 * d, 16)]
        w0b.append(w0d)
        cfb.append((w1d - w0d) * inv_l)

    cp.wait()

    def group_body(g, carry):
        b = g * 16
        acc = jnp.zeros((16,), jnp.int32)
        for l in range(L):
            acc = acc + x_v[l, pl.ds(b, 16)]
        s = acc.astype(jnp.float32)
        for d in range(D):
            out_v[d, pl.ds(b, 16)] = w0b[d] + s * cfb[d]
        return carry

    lax.fori_loop(0, NGROUPS, group_body, 0)
    pltpu.sync_copy(out_v, out_hbm.at[:, pl.ds(b0, BPW)])


def kernel(x, use_quantized, W):
    del use_quantized
    xt = x.astype(jnp.int32).T
    Wf = W.astype(jnp.float32)
    w0 = jnp.broadcast_to(Wf[0][:, None], (D, 16)).reshape(D * 16)
    w1 = jnp.broadcast_to(Wf[1][:, None], (D, 16)).reshape(D * 16)
    out_t = _bag_mean(xt, w0, w1)
    return out_t.T
